# Initial kernel scaffold; baseline (speedup 1.0000x reference)
import jax
import jax.numpy as jnp
from jax.experimental import pallas as pl


def kernel(x, edge_index, edge_attr, W, b):
    raise NotImplementedError("write your pallas kernel here")


# speedup vs baseline: 2.2942x; 2.2942x over previous
import functools

import jax
import jax.numpy as jnp
from jax import lax
from jax.experimental import pallas as pl
from jax.experimental.pallas import tpu as pltpu
from jax.experimental.pallas import tpu_sc as plsc

N_NODES = 10000
N_EDGES = 320000
D = 128
DE = 16
LANES = 16

NC = 2
NS = 16
NW = NC * NS
EPW = N_EDGES // NW
CHUNK = 80
NCHUNK = EPW // CHUNK

_SC_MESH = plsc.VectorSubcoreMesh(
    core_axis_name="c", subcore_axis_name="s", num_cores=NC, num_subcores=NS)


def _tables_body(x_ref, ws_ref, wr_ref, xs_ref, xr_ref):
    xs_ref[...] = jnp.dot(x_ref[...], ws_ref[...],
                          preferred_element_type=jnp.float32)
    xr_ref[...] = jnp.dot(x_ref[...], wr_ref[...],
                          preferred_element_type=jnp.float32)


def _combine_body(g_ref, ea_ref, we_ref, b_ref, o_ref):
    o_ref[...] = (g_ref[...]
                  + jnp.dot(ea_ref[...], we_ref[...],
                            preferred_element_type=jnp.float32)
                  + b_ref[...])


def _gather_sum_body(xs_hbm, xr_hbm, sidx_hbm, ridx_hbm, out_hbm,
                     sidx_v, ridx_v, a_v, b_v, sem_a, sem_b):
    wid = lax.axis_index("s") * NC + lax.axis_index("c")
    base = wid * EPW
    pltpu.sync_copy(sidx_hbm.at[pl.ds(base, EPW)], sidx_v)
    pltpu.sync_copy(ridx_hbm.at[pl.ds(base, EPW)], ridx_v)

    def chunk_body(c, carry):
        off = c * CHUNK
        ca = pltpu.async_copy(xs_hbm.at[sidx_v.at[pl.ds(off, CHUNK)]],
                              a_v, sem_a)
        cb = pltpu.async_copy(xr_hbm.at[ridx_v.at[pl.ds(off, CHUNK)]],
                              b_v, sem_b)
        ca.wait()
        cb.wait()

        def row_body(i, carry2):
            for j in range(D // LANES):
                sl = pl.ds(j * LANES, LANES)
                a_v[i, sl] = a_v[i, sl] + b_v[i, sl]
            return carry2

        lax.fori_loop(0, CHUNK, row_body, 0, unroll=2)
        pltpu.sync_copy(a_v, out_hbm.at[pl.ds(base + off, CHUNK)])
        return carry

    lax.fori_loop(0, NCHUNK, chunk_body, 0)


_gather_sum = pl.kernel(
    _gather_sum_body,
    out_type=jax.ShapeDtypeStruct((N_EDGES, D), jnp.float32),
    mesh=_SC_MESH,
    scratch_types=[
        pltpu.VMEM((EPW,), jnp.int32),
        pltpu.VMEM((EPW,), jnp.int32),
        pltpu.VMEM((CHUNK, D), jnp.float32),
        pltpu.VMEM((CHUNK, D), jnp.float32),
        pltpu.SemaphoreType.DMA,
        pltpu.SemaphoreType.DMA,
    ],
)

_EB = 3200


def kernel(x, edge_index, edge_attr, W, b):
    s_idx = edge_index[0].astype(jnp.int32)
    r_idx = edge_index[1].astype(jnp.int32)
    ws = W[:D]
    wr = W[D:2 * D]
    we = W[2 * D:]
    b2 = b.reshape(1, D)

    xs, xr = pl.pallas_call(
        _tables_body,
        out_shape=[jax.ShapeDtypeStruct((N_NODES, D), jnp.float32)] * 2,
    )(x, ws, wr)

    g = _gather_sum(xs, xr, s_idx, r_idx)

    out = pl.pallas_call(
        _combine_body,
        grid=(N_EDGES // _EB,),
        in_specs=[
            pl.BlockSpec((_EB, D), lambda i: (i, 0)),
            pl.BlockSpec((_EB, DE), lambda i: (i, 0)),
            pl.BlockSpec((DE, D), lambda i: (0, 0)),
            pl.BlockSpec((1, D), lambda i: (0, 0)),
        ],
        out_specs=pl.BlockSpec((_EB, D), lambda i: (i, 0)),
        out_shape=jax.ShapeDtypeStruct((N_EDGES, D), jnp.float32),
    )(g, edge_attr, we, b2)

    return (x, edge_index, out)


# speedup vs baseline: 2.9355x; 1.2795x over previous
import functools

import jax
import jax.numpy as jnp
from jax import lax
from jax.experimental import pallas as pl
from jax.experimental.pallas import tpu as pltpu
from jax.experimental.pallas import tpu_sc as plsc

N_NODES = 10000
N_EDGES = 320000
D = 128
DE = 16
LANES = 16

NC = 2
NS = 16
NW = NC * NS
EPW = N_EDGES // NW
CHUNK = 200
NCHUNK = EPW // CHUNK

_SC_MESH = plsc.VectorSubcoreMesh(
    core_axis_name="c", subcore_axis_name="s", num_cores=NC, num_subcores=NS)


def _tables_body(x_ref, ws_ref, wr_ref, xs_ref, xr_ref):
    xs_ref[...] = jnp.dot(x_ref[...], ws_ref[...],
                          preferred_element_type=jnp.float32)
    xr_ref[...] = jnp.dot(x_ref[...], wr_ref[...],
                          preferred_element_type=jnp.float32)


def _combine_body(g_ref, ea_ref, we_ref, b_ref, o_ref):
    o_ref[...] = (g_ref[...]
                  + jnp.dot(ea_ref[...], we_ref[...],
                            preferred_element_type=jnp.float32)
                  + b_ref[...])


def _gather_sum_body(xs_hbm, xr_hbm, sidx_hbm, ridx_hbm, out_hbm,
                     sidx_v, ridx_v, a0, a1, b0, b1, sa0, sa1, sb0, sb1):
    wid = lax.axis_index("s") * NC + lax.axis_index("c")
    base = wid * EPW
    pltpu.sync_copy(sidx_hbm.at[pl.ds(base, EPW)], sidx_v)
    pltpu.sync_copy(ridx_hbm.at[pl.ds(base, EPW)], ridx_v)

    a = (a0, a1)
    b = (b0, b1)
    sa = (sa0, sa1)
    sb = (sb0, sb1)

    def gather(c, k):
        off = c * CHUNK
        pltpu.async_copy(xs_hbm.at[sidx_v.at[pl.ds(off, CHUNK)]], a[k], sa[k])
        pltpu.async_copy(xr_hbm.at[ridx_v.at[pl.ds(off, CHUNK)]], b[k], sb[k])

    def wait_buf(k):
        pltpu.make_async_copy(xs_hbm.at[pl.ds(0, CHUNK)], a[k], sa[k]).wait()
        pltpu.make_async_copy(xr_hbm.at[pl.ds(0, CHUNK)], b[k], sb[k]).wait()

    gather(0, 0)
    gather(1, 1)

    def pair_body(p, carry):
        for k in range(2):
            c = p * 2 + k
            wait_buf(k)

            def row_body(i, carry2):
                for j in range(D // LANES):
                    sl = pl.ds(j * LANES, LANES)
                    a[k][i, sl] = a[k][i, sl] + b[k][i, sl]
                return carry2

            lax.fori_loop(0, CHUNK, row_body, 0, unroll=2)
            pltpu.sync_copy(a[k], out_hbm.at[pl.ds(base + c * CHUNK, CHUNK)])

            @pl.when(c + 2 < NCHUNK)
            def _():
                gather(c + 2, k)

        return carry

    lax.fori_loop(0, NCHUNK // 2, pair_body, 0)


_gather_sum = pl.kernel(
    _gather_sum_body,
    out_type=jax.ShapeDtypeStruct((N_EDGES, D), jnp.float32),
    mesh=_SC_MESH,
    scratch_types=[
        pltpu.VMEM((EPW,), jnp.int32),
        pltpu.VMEM((EPW,), jnp.int32),
        pltpu.VMEM((CHUNK, D), jnp.float32),
        pltpu.VMEM((CHUNK, D), jnp.float32),
        pltpu.VMEM((CHUNK, D), jnp.float32),
        pltpu.VMEM((CHUNK, D), jnp.float32),
        pltpu.SemaphoreType.DMA,
        pltpu.SemaphoreType.DMA,
        pltpu.SemaphoreType.DMA,
        pltpu.SemaphoreType.DMA,
    ],
)

_EB = 3200


def kernel(x, edge_index, edge_attr, W, b):
    s_idx = edge_index[0].astype(jnp.int32)
    r_idx = edge_index[1].astype(jnp.int32)
    ws = W[:D]
    wr = W[D:2 * D]
    we = W[2 * D:]
    b2 = b.reshape(1, D)

    xs, xr = pl.pallas_call(
        _tables_body,
        out_shape=[jax.ShapeDtypeStruct((N_NODES, D), jnp.float32)] * 2,
    )(x, ws, wr)

    g = _gather_sum(xs, xr, s_idx, r_idx)

    out = pl.pallas_call(
        _combine_body,
        grid=(N_EDGES // _EB,),
        in_specs=[
            pl.BlockSpec((_EB, D), lambda i: (i, 0)),
            pl.BlockSpec((_EB, DE), lambda i: (i, 0)),
            pl.BlockSpec((DE, D), lambda i: (0, 0)),
            pl.BlockSpec((1, D), lambda i: (0, 0)),
        ],
        out_specs=pl.BlockSpec((_EB, D), lambda i: (i, 0)),
        out_shape=jax.ShapeDtypeStruct((N_EDGES, D), jnp.float32),
    )(g, edge_attr, we, b2)

    return (x, edge_index, out)
